# baseline (speedup 1.0000x reference)
import functools
import jax
import jax.numpy as jnp
from jax import lax
from jax.experimental import pallas as pl
from jax.experimental.pallas import tpu as pltpu
from jax.experimental.pallas import tpu_sc as plsc

N = 10000
D = 128
E = 320000
EPS = 1e-5

NW = 32
CHUNK = 128
E_PAD = 327680
NCHUNK = E_PAD // (NW * CHUNK)
TOTAL_CHUNKS = E_PAD // CHUNK
N_PAD = 10240
EBLK = 8
NCHUNK_C0 = 120
NCHUNK_C1 = 40
ROWS_PER_TILE = N_PAD // 16

_mesh = plsc.VectorSubcoreMesh(core_axis_name="c", subcore_axis_name="s")
_sc_params = pltpu.CompilerParams(needs_layout_passes=False)


@functools.partial(
    pl.kernel,
    out_type=jax.ShapeDtypeStruct((NW * N,), jnp.float32),
    mesh=_mesh,
    scratch_types=[
        pltpu.VMEM((NCHUNK, CHUNK), jnp.int32),
        pltpu.VMEM((NCHUNK, CHUNK), jnp.float32),
        pltpu.VMEM((N,), jnp.float32),
    ],
    compiler_params=_sc_params,
)
def _sc_degree(col_hbm, w_hbm, deg_out, col_v, w_v, deg_v):
    cid = lax.axis_index("c")
    sid = lax.axis_index("s")
    wid = sid * 2 + cid

    pltpu.sync_copy(col_hbm.at[pl.ds(wid * NCHUNK, NCHUNK)], col_v)
    pltpu.sync_copy(w_hbm.at[pl.ds(wid * NCHUNK, NCHUNK)], w_v)

    def zero_body(i, carry):
        deg_v[pl.ds(i * 16, 16)] = jnp.zeros((16,), jnp.float32)
        return carry
    lax.fori_loop(0, N // 16, zero_body, 0)

    def chunk_body(c, carry):
        for j in range(CHUNK // 16):
            sl = pl.ds(j * 16, 16)
            idx = col_v[c, sl]
            val = w_v[c, sl]
            plsc.addupdate_scatter(deg_v, [idx], val)
        return carry
    lax.fori_loop(0, NCHUNK, chunk_body, 0)

    pltpu.sync_copy(deg_v, deg_out.at[pl.ds(wid * N, N)])


@functools.partial(
    pl.kernel,
    out_type=jax.ShapeDtypeStruct((2, N_PAD, D), jnp.float32),
    mesh=_mesh,
    scratch_types=[
        pltpu.VMEM((2, EBLK, CHUNK), jnp.int32),
        pltpu.VMEM((2, EBLK, CHUNK), jnp.int32),
        pltpu.VMEM((2, EBLK, CHUNK), jnp.float32),
        pltpu.VMEM((2, CHUNK, D), jnp.float32),
        pltpu.VMEM_SHARED((N_PAD, D), jnp.float32),
        pltpu.SemaphoreType.DMA,
        pltpu.SemaphoreType.DMA,
    ],
    compiler_params=_sc_params,
)
def _sc_scatter(hp_hbm, row_hbm, col_hbm, w_hbm, acc_out,
                row_v, col_v, w_v, rows_v, acc_s, gsem, ssem):
    cid = lax.axis_index("c")
    sid = lax.axis_index("s")

    def zero_body(r, carry):
        for j in range(D // 16):
            rows_v[0, r, pl.ds(j * 16, 16)] = jnp.zeros((16,), jnp.float32)
        return carry
    lax.fori_loop(0, CHUNK, zero_body, 0)
    for k in range(5):
        pltpu.sync_copy(rows_v.at[0],
                        acc_s.at[pl.ds(sid * ROWS_PER_TILE + k * 128, 128)])
    plsc.subcore_barrier()

    def scale_chunk(mb, c, buf):
        def scale_body(b, carry2):
            wv = w_v[mb, c, pl.ds(b * 16, 16)]
            for l in range(16):
                sv = lax.broadcast(wv[l], (16,))
                e = b * 16 + l
                for j in range(D // 16):
                    sl = pl.ds(j * 16, 16)
                    rows_v[buf, e, sl] = rows_v[buf, e, sl] * sv
            return carry2
        lax.fori_loop(0, CHUNK // 16, scale_body, 0)

    def wait_one_scatter():
        pltpu.make_async_copy(rows_v.at[0], acc_s.at[col_v.at[0, 0]],
                              ssem).wait()

    def do_sb(start, mb, first_pred):
        esl = pl.ds(start, EBLK)
        pltpu.sync_copy(row_hbm.at[esl], row_v.at[mb])
        pltpu.sync_copy(col_hbm.at[esl], col_v.at[mb])
        pltpu.sync_copy(w_hbm.at[esl], w_v.at[mb])

        gathers = [None, None]
        gathers[0] = pltpu.async_copy(hp_hbm.at[row_v.at[mb, 0]],
                                      rows_v.at[0], gsem)
        for c in range(EBLK):
            rb = c % 2
            gathers[rb].wait()
            if c == 0 and first_pred is not None:
                @pl.when(first_pred)
                def _():
                    wait_one_scatter()
            else:
                wait_one_scatter()
            if c + 1 < EBLK:
                gathers[1 - rb] = pltpu.async_copy(
                    hp_hbm.at[row_v.at[mb, c + 1]], rows_v.at[1 - rb], gsem)
            scale_chunk(mb, c, rb)
            pltpu.async_copy(rows_v.at[rb],
                             acc_s.at[col_v.at[mb, c]], ssem, add=True)

    def run_core(base, nsb):
        npair = nsb // 2

        def pair_body(i, carry):
            do_sb(base + 2 * i * EBLK, 0, i > 0)
            do_sb(base + (2 * i + 1) * EBLK, 1, None)
            return carry
        lax.fori_loop(0, npair, pair_body, 0)
        if nsb % 2:
            do_sb(base + (nsb - 1) * EBLK, 0, None)
        wait_one_scatter()

    tile_base = sid * (NCHUNK_C0 + NCHUNK_C1)

    @pl.when(cid == 0)
    def _():
        run_core(tile_base, NCHUNK_C0 // EBLK)

    @pl.when(cid == 1)
    def _():
        run_core(tile_base + NCHUNK_C0, NCHUNK_C1 // EBLK)

    plsc.subcore_barrier()
    base = sid * ROWS_PER_TILE
    pltpu.sync_copy(acc_s.at[pl.ds(base, ROWS_PER_TILE)],
                    acc_out.at[cid, pl.ds(base, ROWS_PER_TILE)])


BN_ROWS = 400
NBLK = N // BN_ROWS


def _dinv_of(deg_blk):
    deg = jnp.sum(deg_blk, axis=1)
    return jnp.where(deg > 0, lax.rsqrt(deg), 0.0)


def _bdot(a, b):
    return jnp.dot(a.astype(jnp.bfloat16), b.astype(jnp.bfloat16),
                   preferred_element_type=jnp.float32)


def _tc_pre_kernel(deg_ref, x_ref, wg_ref, wl_ref, hp_ref, hl_ref):
    dinv = _dinv_of(deg_ref[...])
    h = _bdot(x_ref[...], wg_ref[...])
    hp_ref[...] = h * dinv[:, None]
    hl_ref[...] = _bdot(x_ref[...], wl_ref[...])


def _tc_pre(deg_p, x, wg, wl):
    return pl.pallas_call(
        _tc_pre_kernel,
        grid=(NBLK,),
        in_specs=[
            pl.BlockSpec((BN_ROWS, NW), lambda i: (i, 0)),
            pl.BlockSpec((BN_ROWS, D), lambda i: (i, 0)),
            pl.BlockSpec((D, D), lambda i: (0, 0)),
            pl.BlockSpec((D, D), lambda i: (0, 0)),
        ],
        out_specs=[
            pl.BlockSpec((BN_ROWS, D), lambda i: (i, 0)),
            pl.BlockSpec((BN_ROWS, D), lambda i: (i, 0)),
        ],
        out_shape=[jax.ShapeDtypeStruct((N, D), jnp.float32)] * 2,
    )(deg_p, x, wg, wl)


def _tc_post_kernel(hl_ref, acc_ref, deg_ref, gamma_ref, beta_ref,
                    wg_ref, wl_ref, hp_ref, hlo_ref, sum_s, sumsq_s,
                    out_c, dinv_c, *, final):
    p = pl.program_id(0)
    i = pl.program_id(1)

    @pl.when(p == 0)
    def _():
        dinv = _dinv_of(deg_ref[...])
        out = hl_ref[...] + (acc_ref[0] + acc_ref[1]) * dinv[:, None]
        out_c[i] = out
        dinv_c[i] = dinv[:, None]

        @pl.when(i == 0)
        def _():
            sum_s[...] = jnp.zeros_like(sum_s)
            sumsq_s[...] = jnp.zeros_like(sumsq_s)
        sum_s[...] += jnp.sum(out, axis=0, keepdims=True)
        sumsq_s[...] += jnp.sum(out * out, axis=0, keepdims=True)

    @pl.when(p == 1)
    def _():
        out = out_c[i]
        mean = sum_s[...] / N
        var = sumsq_s[...] / N - mean * mean
        xn = (out - mean) * lax.rsqrt(var + EPS) * gamma_ref[...] + beta_ref[...]
        if final:
            hp_ref[...] = xn
            hlo_ref[...] = xn
        else:
            h1 = jnp.maximum(xn, 0.0)
            hp_ref[...] = _bdot(h1, wg_ref[...]) * dinv_c[i]
            hlo_ref[...] = _bdot(h1, wl_ref[...])


def _tc_post(hl, acc, deg_p, gamma, beta, wg, wl, final):
    return pl.pallas_call(
        functools.partial(_tc_post_kernel, final=final),
        grid=(2, NBLK),
        in_specs=[
            pl.BlockSpec((BN_ROWS, D), lambda p, i: ((1 - p) * i, 0)),
            pl.BlockSpec((2, BN_ROWS, D), lambda p, i: (0, (1 - p) * i, 0)),
            pl.BlockSpec((BN_ROWS, NW), lambda p, i: ((1 - p) * i, 0)),
            pl.BlockSpec((1, D), lambda p, i: (0, 0)),
            pl.BlockSpec((1, D), lambda p, i: (0, 0)),
            pl.BlockSpec((D, D), lambda p, i: (0, 0)),
            pl.BlockSpec((D, D), lambda p, i: (0, 0)),
        ],
        out_specs=[
            pl.BlockSpec((BN_ROWS, D), lambda p, i: (i, 0)),
            pl.BlockSpec((BN_ROWS, D), lambda p, i: (i, 0)),
        ],
        out_shape=[jax.ShapeDtypeStruct((N, D), jnp.float32)] * 2,
        scratch_shapes=[
            pltpu.VMEM((1, D), jnp.float32),
            pltpu.VMEM((1, D), jnp.float32),
            pltpu.VMEM((NBLK, BN_ROWS, D), jnp.float32),
            pltpu.VMEM((NBLK, BN_ROWS, 1), jnp.float32),
        ],
    )(hl, acc, deg_p, gamma, beta, wg, wl)


def kernel(x, edge_index, edge_weight, W_lin0, W_gcn0, gamma0, beta0,
           W_lin1, W_gcn1, gamma1, beta1):
    pad = E_PAD - E
    row3 = jnp.pad(edge_index[0], (0, pad)).reshape(TOTAL_CHUNKS, CHUNK)
    col3 = jnp.pad(edge_index[1], (0, pad)).reshape(TOTAL_CHUNKS, CHUNK)
    w3 = jnp.pad(edge_weight, (0, pad)).reshape(TOTAL_CHUNKS, CHUNK)
    g0 = gamma0.reshape(1, D)
    b0 = beta0.reshape(1, D)
    g1 = gamma1.reshape(1, D)
    b1 = beta1.reshape(1, D)

    deg_p = _sc_degree(col3, w3).reshape(NW, N).T

    hp0, hl0 = _tc_pre(deg_p, x, W_gcn0, W_lin0)
    acc0 = _sc_scatter(hp0, row3, col3, w3)
    hp1, hl1 = _tc_post(hl0, acc0, deg_p, g0, b0, W_gcn1, W_lin1, final=False)
    acc1 = _sc_scatter(hp1, row3, col3, w3)
    out, _ = _tc_post(hl1, acc1, deg_p, g1, b1, W_gcn1, W_lin1, final=True)
    return out


# speedup vs baseline: 1.0103x; 1.0103x over previous
import functools
import jax
import jax.numpy as jnp
from jax import lax
from jax.experimental import pallas as pl
from jax.experimental.pallas import tpu as pltpu
from jax.experimental.pallas import tpu_sc as plsc

N = 10000
D = 128
E = 320000
EPS = 1e-5

NW = 32
CHUNK = 128
E_PAD = 327680
NCHUNK = E_PAD // (NW * CHUNK)
TOTAL_CHUNKS = E_PAD // CHUNK
N_PAD = 10240
EBLK = 8
NCHUNK_C0 = 120
NCHUNK_C1 = 40
ROWS_PER_TILE = N_PAD // 16

_mesh = plsc.VectorSubcoreMesh(core_axis_name="c", subcore_axis_name="s")
_sc_params = pltpu.CompilerParams(needs_layout_passes=False)


@functools.partial(
    pl.kernel,
    out_type=jax.ShapeDtypeStruct((NW * N,), jnp.float32),
    mesh=_mesh,
    scratch_types=[
        pltpu.VMEM((NCHUNK, CHUNK), jnp.int32),
        pltpu.VMEM((NCHUNK, CHUNK), jnp.float32),
        pltpu.VMEM((N,), jnp.float32),
    ],
    compiler_params=_sc_params,
)
def _sc_degree(col_hbm, w_hbm, deg_out, col_v, w_v, deg_v):
    cid = lax.axis_index("c")
    sid = lax.axis_index("s")
    wid = sid * 2 + cid

    pltpu.sync_copy(col_hbm.at[pl.ds(wid * NCHUNK, NCHUNK)], col_v)
    pltpu.sync_copy(w_hbm.at[pl.ds(wid * NCHUNK, NCHUNK)], w_v)

    def zero_body(i, carry):
        deg_v[pl.ds(i * 16, 16)] = jnp.zeros((16,), jnp.float32)
        return carry
    lax.fori_loop(0, N // 16, zero_body, 0)

    def chunk_body(c, carry):
        for j in range(CHUNK // 16):
            sl = pl.ds(j * 16, 16)
            idx = col_v[c, sl]
            val = w_v[c, sl]
            plsc.addupdate_scatter(deg_v, [idx], val)
        return carry
    lax.fori_loop(0, NCHUNK, chunk_body, 0)

    pltpu.sync_copy(deg_v, deg_out.at[pl.ds(wid * N, N)])


@functools.partial(
    pl.kernel,
    out_type=jax.ShapeDtypeStruct((2, N_PAD, D), jnp.float32),
    mesh=_mesh,
    scratch_types=[
        pltpu.VMEM((2, EBLK, CHUNK), jnp.int32),
        pltpu.VMEM((2, EBLK, CHUNK), jnp.int32),
        pltpu.VMEM((2, EBLK, CHUNK), jnp.float32),
        pltpu.VMEM((2, CHUNK, D), jnp.float32),
        pltpu.VMEM_SHARED((N_PAD, D), jnp.float32),
        pltpu.SemaphoreType.DMA,
        pltpu.SemaphoreType.DMA,
    ],
    compiler_params=_sc_params,
)
def _sc_scatter(hp_hbm, row_hbm, col_hbm, w_hbm, acc_out,
                row_v, col_v, w_v, rows_v, acc_s, gsem, ssem):
    cid = lax.axis_index("c")
    sid = lax.axis_index("s")

    def zero_body(r, carry):
        for j in range(D // 16):
            rows_v[0, r, pl.ds(j * 16, 16)] = jnp.zeros((16,), jnp.float32)
        return carry
    lax.fori_loop(0, CHUNK, zero_body, 0)
    for k in range(5):
        pltpu.sync_copy(rows_v.at[0],
                        acc_s.at[pl.ds(sid * ROWS_PER_TILE + k * 128, 128)])
    plsc.subcore_barrier()

    def scale_chunk(mb, c, buf):
        def scale_body(b, carry2):
            wv = w_v[mb, c, pl.ds(b * 16, 16)]
            for l in range(16):
                sv = lax.broadcast(wv[l], (16,))
                e = b * 16 + l
                for j in range(D // 16):
                    sl = pl.ds(j * 16, 16)
                    rows_v[buf, e, sl] = rows_v[buf, e, sl] * sv
            return carry2
        lax.fori_loop(0, CHUNK // 16, scale_body, 0)

    def wait_one_scatter():
        pltpu.make_async_copy(rows_v.at[0], acc_s.at[col_v.at[0, 0]],
                              ssem).wait()

    def do_sb(start, mb, first_pred):
        esl = pl.ds(start, EBLK)
        pltpu.sync_copy(row_hbm.at[esl], row_v.at[mb])
        pltpu.sync_copy(col_hbm.at[esl], col_v.at[mb])
        pltpu.sync_copy(w_hbm.at[esl], w_v.at[mb])

        gathers = [None, None]
        gathers[0] = pltpu.async_copy(hp_hbm.at[row_v.at[mb, 0]],
                                      rows_v.at[0], gsem)
        for c in range(EBLK):
            rb = c % 2
            gathers[rb].wait()
            if c == 0 and first_pred is not None:
                @pl.when(first_pred)
                def _():
                    wait_one_scatter()
            else:
                wait_one_scatter()
            if c + 1 < EBLK:
                gathers[1 - rb] = pltpu.async_copy(
                    hp_hbm.at[row_v.at[mb, c + 1]], rows_v.at[1 - rb], gsem)
            scale_chunk(mb, c, rb)
            pltpu.async_copy(rows_v.at[rb],
                             acc_s.at[col_v.at[mb, c]], ssem, add=True)

    def run_core(base, nsb):
        npair = nsb // 2

        def pair_body(i, carry):
            do_sb(base + 2 * i * EBLK, 0, i > 0)
            do_sb(base + (2 * i + 1) * EBLK, 1, None)
            return carry
        lax.fori_loop(0, npair, pair_body, 0)
        if nsb % 2:
            do_sb(base + (nsb - 1) * EBLK, 0, None)
        wait_one_scatter()

    tile_base = sid * (NCHUNK_C0 + NCHUNK_C1)

    @pl.when(cid == 0)
    def _():
        run_core(tile_base, NCHUNK_C0 // EBLK)

    @pl.when(cid == 1)
    def _():
        run_core(tile_base + NCHUNK_C0, NCHUNK_C1 // EBLK)

    plsc.subcore_barrier()
    base = sid * ROWS_PER_TILE
    pltpu.sync_copy(acc_s.at[pl.ds(base, ROWS_PER_TILE)],
                    acc_out.at[cid, pl.ds(base, ROWS_PER_TILE)])


BN_ROWS = 400
NBLK = N // BN_ROWS


def _dinv_of(deg_blk):
    deg = jnp.sum(deg_blk, axis=1)
    return jnp.where(deg > 0, lax.rsqrt(deg), 0.0)


def _tc_pre_kernel(deg_ref, x_ref, wg_ref, wl_ref, hp_ref, hl_ref):
    dinv = _dinv_of(deg_ref[...])
    h = jnp.dot(x_ref[...], wg_ref[...], preferred_element_type=jnp.float32)
    hp_ref[...] = h * dinv[:, None]
    hl_ref[...] = jnp.dot(x_ref[...], wl_ref[...],
                          preferred_element_type=jnp.float32)


def _tc_pre(deg_p, x, wg, wl):
    return pl.pallas_call(
        _tc_pre_kernel,
        grid=(NBLK,),
        in_specs=[
            pl.BlockSpec((BN_ROWS, NW), lambda i: (i, 0)),
            pl.BlockSpec((BN_ROWS, D), lambda i: (i, 0)),
            pl.BlockSpec((D, D), lambda i: (0, 0)),
            pl.BlockSpec((D, D), lambda i: (0, 0)),
        ],
        out_specs=[
            pl.BlockSpec((BN_ROWS, D), lambda i: (i, 0)),
            pl.BlockSpec((BN_ROWS, D), lambda i: (i, 0)),
        ],
        out_shape=[jax.ShapeDtypeStruct((N, D), jnp.float32)] * 2,
    )(deg_p, x, wg, wl)


def _tc_post_kernel(hl_ref, acc_ref, deg_ref, gamma_ref, beta_ref,
                    wg_ref, wl_ref, hp_ref, hlo_ref, sum_s, sumsq_s,
                    out_c, dinv_c, *, final):
    p = pl.program_id(0)
    i = pl.program_id(1)

    @pl.when(p == 0)
    def _():
        dinv = _dinv_of(deg_ref[...])
        out = hl_ref[...] + (acc_ref[0] + acc_ref[1]) * dinv[:, None]
        out_c[i] = out
        dinv_c[i] = dinv[:, None]

        @pl.when(i == 0)
        def _():
            sum_s[...] = jnp.zeros_like(sum_s)
            sumsq_s[...] = jnp.zeros_like(sumsq_s)
        sum_s[...] += jnp.sum(out, axis=0, keepdims=True)
        sumsq_s[...] += jnp.sum(out * out, axis=0, keepdims=True)

    @pl.when(p == 1)
    def _():
        out = out_c[i]
        mean = sum_s[...] / N
        var = sumsq_s[...] / N - mean * mean
        xn = (out - mean) * lax.rsqrt(var + EPS) * gamma_ref[...] + beta_ref[...]
        if final:
            hp_ref[...] = xn
            hlo_ref[...] = xn
        else:
            h1 = jnp.maximum(xn, 0.0)
            hp_ref[...] = jnp.dot(h1, wg_ref[...],
                                  preferred_element_type=jnp.float32) * dinv_c[i]
            hlo_ref[...] = jnp.dot(h1, wl_ref[...],
                                   preferred_element_type=jnp.float32)


def _tc_post(hl, acc, deg_p, gamma, beta, wg, wl, final):
    return pl.pallas_call(
        functools.partial(_tc_post_kernel, final=final),
        grid=(2, NBLK),
        in_specs=[
            pl.BlockSpec((BN_ROWS, D), lambda p, i: ((1 - p) * i, 0)),
            pl.BlockSpec((2, BN_ROWS, D), lambda p, i: (0, (1 - p) * i, 0)),
            pl.BlockSpec((BN_ROWS, NW), lambda p, i: ((1 - p) * i, 0)),
            pl.BlockSpec((1, D), lambda p, i: (0, 0)),
            pl.BlockSpec((1, D), lambda p, i: (0, 0)),
            pl.BlockSpec((D, D), lambda p, i: (0, 0)),
            pl.BlockSpec((D, D), lambda p, i: (0, 0)),
        ],
        out_specs=[
            pl.BlockSpec((BN_ROWS, D), lambda p, i: (i, 0)),
            pl.BlockSpec((BN_ROWS, D), lambda p, i: (i, 0)),
        ],
        out_shape=[jax.ShapeDtypeStruct((N, D), jnp.float32)] * 2,
        scratch_shapes=[
            pltpu.VMEM((1, D), jnp.float32),
            pltpu.VMEM((1, D), jnp.float32),
            pltpu.VMEM((NBLK, BN_ROWS, D), jnp.float32),
            pltpu.VMEM((NBLK, BN_ROWS, 1), jnp.float32),
        ],
    )(hl, acc, deg_p, gamma, beta, wg, wl)


def kernel(x, edge_index, edge_weight, W_lin0, W_gcn0, gamma0, beta0,
           W_lin1, W_gcn1, gamma1, beta1):
    pad = E_PAD - E
    row3 = jnp.pad(edge_index[0], (0, pad)).reshape(TOTAL_CHUNKS, CHUNK)
    col3 = jnp.pad(edge_index[1], (0, pad)).reshape(TOTAL_CHUNKS, CHUNK)
    w3 = jnp.pad(edge_weight, (0, pad)).reshape(TOTAL_CHUNKS, CHUNK)
    g0 = gamma0.reshape(1, D)
    b0 = beta0.reshape(1, D)
    g1 = gamma1.reshape(1, D)
    b1 = beta1.reshape(1, D)

    deg_p = _sc_degree(col3, w3).reshape(NW, N).T

    hp0, hl0 = _tc_pre(deg_p, x, W_gcn0, W_lin0)
    acc0 = _sc_scatter(hp0, row3, col3, w3)
    hp1, hl1 = _tc_post(hl0, acc0, deg_p, g0, b0, W_gcn1, W_lin1, final=False)
    acc1 = _sc_scatter(hp1, row3, col3, w3)
    out, _ = _tc_post(hl1, acc1, deg_p, g1, b1, W_gcn1, W_lin1, final=True)
    return out
